# baseline (speedup 1.0000x reference)
import functools

import jax
import jax.numpy as jnp
import numpy as np
from jax.experimental import pallas as pl
from jax.experimental.pallas import tpu as pltpu

_EOS = 3
_NCMD = 6
_NARGS = 16
_ADIM = 257
_EW3 = float(np.exp(-6.0))


def _args_body(t2_ref, cmd2_ref, x_ref, na_ref, da_ref):
    i = pl.program_id(0)
    xb = x_ref[...]
    rows = xb.shape[0] * xb.shape[1] * xb.shape[2]
    x = xb.reshape(rows, _ADIM)
    t = t2_ref[...]

    ones8 = (jax.lax.broadcasted_iota(jnp.int32, (_ADIM, 8), 0) >= 0
             ).astype(jnp.float32)

    e = jnp.exp(x)
    c = jax.lax.broadcasted_iota(jnp.int32, x.shape, 1)
    ad = jnp.abs(c - t)
    w = jnp.where(ad <= 3, jnp.exp(-2.0 * ad.astype(jnp.float32)), 0.0)
    wx = w * x

    s = jax.lax.dot(e, ones8, preferred_element_type=jnp.float32)[:, 0:1]
    z = jax.lax.dot(w, ones8, preferred_element_type=jnp.float32)[:, 0:1]
    g = jax.lax.dot(wx, ones8, preferred_element_type=jnp.float32)[:, 0:1]

    tf = t.astype(jnp.float32)
    delta = jnp.where(t >= 253, _EW3 - jnp.exp(-2.0 * (256.0 - tf)), 0.0)
    z = z + delta
    g = g + delta * x[:, 256:257]

    cmdb = cmd2_ref[...]
    a = jax.lax.broadcasted_iota(jnp.int32, (rows, 1), 0) & 15
    mask = (((cmdb == 0) & (a < 2)) |
            ((cmdb == 1) & (a < 4)) |
            ((cmdb == 2) & ((a < 2) | (a == 4))) |
            ((cmdb == 5) & (a >= 5))).astype(jnp.float32)

    la = jnp.sum(mask * (jnp.log(s) - g / z))
    da = jnp.sum(mask)

    @pl.when(i == 0)
    def _():
        na_ref[0, 0] = la
        da_ref[0, 0] = da

    @pl.when(i != 0)
    def _():
        na_ref[0, 0] += la
        da_ref[0, 0] += da


def _cmd_body(cl_ref, cmd_ref, nc_ref, dc_ref):
    cl = cl_ref[...]
    cmdf = cmd_ref[...]
    eos = (cmdf == _EOS).astype(jnp.float32)
    sdim = cmdf.shape[1]
    r = jax.lax.broadcasted_iota(jnp.int32, (sdim, sdim), 0)
    cc = jax.lax.broadcasted_iota(jnp.int32, (sdim, sdim), 1)
    tri = (r < cc).astype(jnp.float32)
    excl = jnp.dot(eos, tri, preferred_element_type=jnp.float32)
    pad0 = (excl == 0.0).astype(jnp.float32)
    vis = (jnp.sum(eos, axis=1) < float(sdim)).astype(jnp.float32)
    pad = pad0 * vis[:, None]
    mx = jnp.max(cl, axis=-1)
    lse6 = mx + jnp.log(jnp.sum(jnp.exp(cl - mx[..., None]), axis=-1))
    c6 = jax.lax.broadcasted_iota(jnp.int32, cl.shape, 2)
    picked = jnp.sum(jnp.where(c6 == cmdf[..., None], cl, 0.0), axis=-1)
    nll = lse6 - picked
    nc_ref[0, 0] = jnp.sum(pad * nll)
    dc_ref[0, 0] = jnp.sum(pad)


@jax.jit
def kernel(command_logits, args_logits, command, args):
    bsz, sdim = command.shape
    p = bsz * sdim * _NARGS
    bb = 4
    rows = bb * sdim * _NARGS
    grid = bsz // bb
    t2 = args.reshape(p, 1) + 1
    cmd2 = jnp.broadcast_to(command.reshape(bsz * sdim, 1),
                            (bsz * sdim, _NARGS)).reshape(p, 1)

    scalar_spec = pl.BlockSpec((1, 1), lambda i: (0, 0),
                               memory_space=pltpu.SMEM)
    na, da = pl.pallas_call(
        _args_body,
        grid=(grid,),
        in_specs=[
            pl.BlockSpec((rows, 1), lambda i: (i, 0)),
            pl.BlockSpec((rows, 1), lambda i: (i, 0)),
            pl.BlockSpec((bb, sdim, _NARGS, _ADIM), lambda i: (i, 0, 0, 0)),
        ],
        out_specs=[scalar_spec] * 2,
        out_shape=[jax.ShapeDtypeStruct((1, 1), jnp.float32)] * 2,
        compiler_params=pltpu.CompilerParams(
            dimension_semantics=("arbitrary",)),
    )(t2, cmd2, args_logits)

    scalar_spec0 = pl.BlockSpec((1, 1), lambda: (0, 0),
                                memory_space=pltpu.SMEM)
    nc, dc = pl.pallas_call(
        _cmd_body,
        out_specs=[scalar_spec0] * 2,
        out_shape=[jax.ShapeDtypeStruct((1, 1), jnp.float32)] * 2,
    )(command_logits, command)

    loss_cmd = nc[0, 0] / dc[0, 0]
    loss_args = 2.0 * na[0, 0] / da[0, 0]
    return (loss_cmd, loss_args)


# speedup vs baseline: 1.2159x; 1.2159x over previous
import functools

import jax
import jax.numpy as jnp
import numpy as np
from jax.experimental import pallas as pl
from jax.experimental.pallas import tpu as pltpu

_EOS = 3
_NCMD = 6
_NARGS = 16
_ADIM = 257
_EW3 = float(np.exp(-6.0))


def _args_body(t_ref, cmd_ref, x_ref, na_ref, da_ref):
    i = pl.program_id(0)
    x = x_ref[...]
    t = t_ref[...] + 1

    e = jnp.exp(x)
    s = jnp.sum(e, axis=-1)

    c = jax.lax.broadcasted_iota(jnp.int32, x.shape, 3)
    ad = jnp.abs(c - t[..., None])
    w = jnp.where(ad <= 3, jnp.exp(-2.0 * ad.astype(jnp.float32)), 0.0)
    z = jnp.sum(w, axis=-1)
    g = jnp.sum(w * x, axis=-1)

    tf = t.astype(jnp.float32)
    delta = jnp.where(t >= 253, _EW3 - jnp.exp(-2.0 * (256.0 - tf)), 0.0)
    z = z + delta
    g = g + delta * x[..., 256]

    cmdb = cmd_ref[...][0][..., None]
    a = jax.lax.broadcasted_iota(jnp.int32, t.shape, 2)
    mask = (((cmdb == 0) & (a < 2)) |
            ((cmdb == 1) & (a < 4)) |
            ((cmdb == 2) & ((a < 2) | (a == 4))) |
            ((cmdb == 5) & (a >= 5))).astype(jnp.float32)

    la = jnp.sum(mask * (jnp.log(s) - g / z))
    da = jnp.sum(mask)

    @pl.when(i == 0)
    def _():
        na_ref[0, 0] = la
        da_ref[0, 0] = da

    @pl.when(i != 0)
    def _():
        na_ref[0, 0] += la
        da_ref[0, 0] += da


def _cmd_body(cl_ref, cmd_ref, nc_ref, dc_ref):
    cl = cl_ref[...]
    cmdf = cmd_ref[...]
    eos = (cmdf == _EOS).astype(jnp.float32)
    sdim = cmdf.shape[1]
    r = jax.lax.broadcasted_iota(jnp.int32, (sdim, sdim), 0)
    cc = jax.lax.broadcasted_iota(jnp.int32, (sdim, sdim), 1)
    tri = (r < cc).astype(jnp.float32)
    excl = jnp.dot(eos, tri, preferred_element_type=jnp.float32)
    pad0 = (excl == 0.0).astype(jnp.float32)
    vis = (jnp.sum(eos, axis=1) < float(sdim)).astype(jnp.float32)
    pad = pad0 * vis[:, None]
    mx = jnp.max(cl, axis=-1)
    lse6 = mx + jnp.log(jnp.sum(jnp.exp(cl - mx[..., None]), axis=-1))
    c6 = jax.lax.broadcasted_iota(jnp.int32, cl.shape, 2)
    picked = jnp.sum(jnp.where(c6 == cmdf[..., None], cl, 0.0), axis=-1)
    nll = lse6 - picked
    nc_ref[0, 0] = jnp.sum(pad * nll)
    dc_ref[0, 0] = jnp.sum(pad)


@jax.jit
def kernel(command_logits, args_logits, command, args):
    bsz, sdim = command.shape
    bb = 4
    grid = bsz // bb

    scalar_spec = pl.BlockSpec((1, 1), lambda i: (0, 0),
                               memory_space=pltpu.SMEM)
    na, da = pl.pallas_call(
        _args_body,
        grid=(grid,),
        in_specs=[
            pl.BlockSpec((bb, sdim, _NARGS), lambda i: (i, 0, 0)),
            pl.BlockSpec((1, bb, sdim), lambda i: (i, 0, 0)),
            pl.BlockSpec((bb, sdim, _NARGS, _ADIM), lambda i: (i, 0, 0, 0)),
        ],
        out_specs=[scalar_spec] * 2,
        out_shape=[jax.ShapeDtypeStruct((1, 1), jnp.float32)] * 2,
        compiler_params=pltpu.CompilerParams(
            dimension_semantics=("arbitrary",)),
    )(args, command.reshape(grid, bb, sdim), args_logits)

    scalar_spec0 = pl.BlockSpec((1, 1), lambda: (0, 0),
                                memory_space=pltpu.SMEM)
    nc, dc = pl.pallas_call(
        _cmd_body,
        out_specs=[scalar_spec0] * 2,
        out_shape=[jax.ShapeDtypeStruct((1, 1), jnp.float32)] * 2,
    )(command_logits, command)

    loss_cmd = nc[0, 0] / dc[0, 0]
    loss_args = 2.0 * na[0, 0] / da[0, 0]
    return (loss_cmd, loss_args)
